# Initial kernel scaffold; baseline (speedup 1.0000x reference)
import jax
import jax.numpy as jnp
from jax.experimental import pallas as pl


def kernel(x_track, x_playlist, edge_src, edge_dst, label_src, label_dst, W_lt, b_lt, W_lp, b_lp, Wrel_c0, Wroot_c0, b_c0, Wrel_r0, Wroot_r0, b_r0, Wrel_c1, Wroot_c1, b_c1, Wrel_r1, Wroot_r1, b_r1):
    raise NotImplementedError("write your pallas kernel here")


# speedup vs baseline: 2.1810x; 2.1810x over previous
import functools

import jax
import jax.numpy as jnp
from jax import lax
from jax.experimental import pallas as pl
from jax.experimental.pallas import tpu as pltpu
from jax.experimental.pallas import tpu_sc as plsc

N_TRACK = 50000
N_PLAYLIST = 10000
E = 600000
E_LBL = 100000
D = 128
DQ = 32

CHUNK = 1024
GPI = 128
EPAD = 622592
CHUNKS_A = 19
CHUNKS_B = 38
LPAD = 102400

ACC_P = 10240
QACC = 50048


def _zero_acc(zblk, rbuf, acc, s, rows_per_sub, blk):
    pltpu.sync_copy(zblk, rbuf.at[pl.ds(0, blk)])
    n_full = rows_per_sub // blk
    rem = rows_per_sub - n_full * blk
    base = s * rows_per_sub
    for j in range(n_full):
        pltpu.sync_copy(rbuf.at[pl.ds(0, blk)],
                        acc.at[pl.ds(base + j * blk, blk)])
    if rem:
        pltpu.sync_copy(rbuf.at[pl.ds(0, rem)],
                        acc.at[pl.ds(base + n_full * blk, rem)])


def _gs_chunk(table, acc, gbuf, dbuf, rbuf, sem, inflight):
    for h in range(8 // inflight):
        handles = []
        for j in range(inflight):
            handles.append(pltpu.make_async_copy(
                table.at[gbuf.at[h * inflight + j]],
                rbuf.at[pl.ds(j * GPI, GPI)], sem))
        for hd in handles:
            hd.start()
        for hd in handles:
            hd.wait()
        for j in range(inflight):
            pltpu.sync_copy(rbuf.at[pl.ds(j * GPI, GPI)],
                            acc.at[dbuf.at[h * inflight + j]], add=True)


def _seg_partial_call(table, gidx2, didx2, zblk):
    mesh = plsc.VectorSubcoreMesh(core_axis_name="c", subcore_axis_name="s")

    @functools.partial(
        pl.kernel, mesh=mesh,
        out_type=jax.ShapeDtypeStruct((2 * N_PLAYLIST, D), jnp.float32),
        scratch_types=[
            pltpu.VMEM((8, GPI), jnp.int32),
            pltpu.VMEM((8, GPI), jnp.int32),
            pltpu.VMEM((256, D), jnp.float32),
            pltpu.VMEM_SHARED((ACC_P, D), jnp.float32),
            pltpu.SemaphoreType.DMA,
        ],
    )
    def k(table_h, gidx_h, didx_h, zblk_h, out_h, gbuf, dbuf, rbuf, acc, sem):
        c = lax.axis_index("c")
        s = lax.axis_index("s")
        _zero_acc(zblk_h, rbuf, acc, s, ACC_P // 16, 128)
        plsc.subcore_barrier()

        first = (c * 16 + s) * CHUNKS_A

        def body(i, carry):
            row0 = (first + i) * 8
            pltpu.sync_copy(gidx_h.at[pl.ds(row0, 8)], gbuf)
            pltpu.sync_copy(didx_h.at[pl.ds(row0, 8)], dbuf)
            _gs_chunk(table_h, acc, gbuf, dbuf, rbuf, sem, 2)
            return carry

        lax.fori_loop(0, CHUNKS_A, body, 0)
        plsc.subcore_barrier()

        for j in range(8):
            cid = j * 16 + s
            @pl.when(cid < 125)
            def _copy():
                r0 = cid * 80
                pltpu.sync_copy(acc.at[pl.ds(r0, 80)], rbuf.at[pl.ds(0, 80)])
                pltpu.sync_copy(rbuf.at[pl.ds(0, 80)],
                                out_h.at[pl.ds(c * N_PLAYLIST + r0, 80)])

    return k(table, gidx2, didx2, zblk)


def _seg_quarters_call(tableq, gidx2, didx2, zblkq):
    mesh = plsc.VectorSubcoreMesh(core_axis_name="c", subcore_axis_name="s")

    @functools.partial(
        pl.kernel, mesh=mesh,
        compiler_params=pltpu.CompilerParams(use_tc_tiling_on_sc=False),
        out_type=jax.ShapeDtypeStruct((4 * QACC, DQ), jnp.float32),
        scratch_types=[
            pltpu.VMEM((8, GPI), jnp.int32),
            pltpu.VMEM((8, GPI), jnp.int32),
            pltpu.VMEM((512, DQ), jnp.float32),
            pltpu.VMEM_SHARED((QACC, DQ), jnp.float32),
            pltpu.SemaphoreType.DMA,
        ],
    )
    def k(table_h, gidx_h, didx_h, zblk_h, out_h, gbuf, dbuf, rbuf, acc, sem):
        c = lax.axis_index("c")
        s = lax.axis_index("s")
        rows_per_sub = QACC // 16

        for b in range(2):
            q = 2 * c + b
            off = q * N_PLAYLIST

            _zero_acc(zblk_h, rbuf, acc, s, rows_per_sub, 512)
            plsc.subcore_barrier()

            def body(i, carry):
                row0 = (s * CHUNKS_B + i) * 8
                pltpu.sync_copy(gidx_h.at[pl.ds(row0, 8)], gbuf)
                pltpu.sync_copy(didx_h.at[pl.ds(row0, 8)], dbuf)
                for j in range(8):
                    for kk in range(8):
                        v = gbuf[j, pl.ds(kk * 16, 16)]
                        gbuf[j, pl.ds(kk * 16, 16)] = v + off
                _gs_chunk(table_h, acc, gbuf, dbuf, rbuf, sem, 4)
                return carry

            lax.fori_loop(0, CHUNKS_B, body, 0)
            plsc.subcore_barrier()

            base = s * rows_per_sub
            for j in range(6):
                pltpu.sync_copy(acc.at[pl.ds(base + j * 512, 512)], rbuf)
                pltpu.sync_copy(
                    rbuf, out_h.at[pl.ds(q * QACC + base + j * 512, 512)])
            pltpu.sync_copy(acc.at[pl.ds(base + 3072, 56)],
                            rbuf.at[pl.ds(0, 56)])
            pltpu.sync_copy(rbuf.at[pl.ds(0, 56)],
                            out_h.at[pl.ds(q * QACC + base + 3072, 56)])
            plsc.subcore_barrier()

    return k(tableq, gidx2, didx2, zblkq)


def _pred_call(xt, xp, lsrc, ldst):
    mesh = plsc.VectorSubcoreMesh(core_axis_name="c", subcore_axis_name="s")

    @functools.partial(
        pl.kernel, mesh=mesh,
        compiler_params=pltpu.CompilerParams(needs_layout_passes=False),
        out_type=jax.ShapeDtypeStruct((LPAD,), jnp.float32),
        scratch_types=[
            pltpu.VMEM((GPI,), jnp.int32),
            pltpu.VMEM((GPI,), jnp.int32),
            pltpu.VMEM((GPI, D), jnp.float32),
            pltpu.VMEM((GPI, D), jnp.float32),
            pltpu.VMEM((GPI,), jnp.float32),
            pltpu.SemaphoreType.DMA,
        ],
    )
    def k(xt_h, xp_h, ls_h, ld_h, out_h, sbuf, dbuf, ra, rb, obuf, sem):
        c = lax.axis_index("c")
        s = lax.axis_index("s")
        wid = c * 16 + s

        def body(i, carry):
            row = wid * 25 + i
            pltpu.sync_copy(ls_h.at[pl.ds(row * GPI, GPI)], sbuf)
            pltpu.sync_copy(ld_h.at[pl.ds(row * GPI, GPI)], dbuf)
            ha = pltpu.make_async_copy(xt_h.at[sbuf], ra, sem)
            hb = pltpu.make_async_copy(xp_h.at[dbuf], rb, sem)
            ha.start(); hb.start()
            ha.wait(); hb.wait()

            lane = lax.iota(jnp.int32, 16)

            def dot_group(g, carry2):
                accvec = jnp.zeros((16,), jnp.float32)
                for rr in range(16):
                    r = g * 16 + rr
                    acc = ra[r, pl.ds(0, 16)] * rb[r, pl.ds(0, 16)]
                    for kk in range(1, 8):
                        acc = acc + (ra[r, pl.ds(kk * 16, 16)]
                                     * rb[r, pl.ds(kk * 16, 16)])
                    accvec = jnp.where(lane == rr, jnp.sum(acc), accvec)
                obuf[pl.ds(g * 16, 16)] = accvec
                return carry2

            lax.fori_loop(0, 8, dot_group, 0)
            pltpu.sync_copy(obuf, out_h.at[pl.ds(row * GPI, GPI)])
            return carry

        lax.fori_loop(0, 25, body, 0)

    return k(xt, xp, lsrc, ldst)


_RB = 2000


def _lin1(x, w, b):
    n = x.shape[0]

    def body(x_r, w_r, b_r, o_r):
        o_r[...] = jax.lax.dot_general(
            x_r[...], w_r[...], (((1,), (0,)), ((), ())),
            preferred_element_type=jnp.float32) + b_r[...]

    return pl.pallas_call(
        body,
        grid=(n // _RB,),
        in_specs=[
            pl.BlockSpec((_RB, D), lambda i: (i, 0)),
            pl.BlockSpec((D, D), lambda i: (0, 0)),
            pl.BlockSpec((1, D), lambda i: (0, 0)),
        ],
        out_specs=pl.BlockSpec((_RB, D), lambda i: (i, 0)),
        out_shape=jax.ShapeDtypeStruct((n, D), jnp.float32),
    )(x, w, b.reshape(1, D))


def _comb2q(aggq, x, w4, w2, b):
    n = x.shape[0]

    def body(a_r, x_r, w4_r, w2_r, b_r, o_r):
        mm = jax.lax.dot_general(
            x_r[...], w2_r[...], (((1,), (0,)), ((), ())),
            preferred_element_type=jnp.float32)
        for q in range(4):
            mm = mm + jax.lax.dot_general(
                a_r[q], w4_r[q], (((1,), (0,)), ((), ())),
                preferred_element_type=jnp.float32)
        o_r[...] = mm + b_r[...]

    return pl.pallas_call(
        body,
        grid=(n // _RB,),
        in_specs=[
            pl.BlockSpec((4, _RB, DQ), lambda i: (0, i, 0)),
            pl.BlockSpec((_RB, D), lambda i: (i, 0)),
            pl.BlockSpec((4, DQ, D), lambda i: (0, 0, 0)),
            pl.BlockSpec((D, D), lambda i: (0, 0)),
            pl.BlockSpec((1, D), lambda i: (0, 0)),
        ],
        out_specs=pl.BlockSpec((_RB, D), lambda i: (i, 0)),
        out_shape=jax.ShapeDtypeStruct((n, D), jnp.float32),
    )(aggq, x, w4, w2, b.reshape(1, D))


def _comb3(p0, p1, x, w1, w2, b):
    n = x.shape[0]

    def body(p0_r, p1_r, x_r, w1_r, w2_r, b_r, o_r):
        a = p0_r[...] + p1_r[...]
        mm = jax.lax.dot_general(
            a, w1_r[...], (((1,), (0,)), ((), ())),
            preferred_element_type=jnp.float32)
        mm = mm + jax.lax.dot_general(
            x_r[...], w2_r[...], (((1,), (0,)), ((), ())),
            preferred_element_type=jnp.float32)
        o_r[...] = mm + b_r[...]

    return pl.pallas_call(
        body,
        grid=(n // _RB,),
        in_specs=[
            pl.BlockSpec((_RB, D), lambda i: (i, 0)),
            pl.BlockSpec((_RB, D), lambda i: (i, 0)),
            pl.BlockSpec((_RB, D), lambda i: (i, 0)),
            pl.BlockSpec((D, D), lambda i: (0, 0)),
            pl.BlockSpec((D, D), lambda i: (0, 0)),
            pl.BlockSpec((1, D), lambda i: (0, 0)),
        ],
        out_specs=pl.BlockSpec((_RB, D), lambda i: (i, 0)),
        out_shape=jax.ShapeDtypeStruct((n, D), jnp.float32),
    )(p0, p1, x, w1, w2, b.reshape(1, D))


def kernel(x_track, x_playlist, edge_src, edge_dst, label_src, label_dst,
           W_lt, b_lt, W_lp, b_lp,
           Wrel_c0, Wroot_c0, b_c0, Wrel_r0, Wroot_r0, b_r0,
           Wrel_c1, Wroot_c1, b_c1, Wrel_r1, Wroot_r1, b_r1):
    ep = EPAD - E
    g_p = jnp.pad(edge_src, (0, ep)).reshape(-1, GPI)
    s_p = jnp.pad(edge_dst, (0, ep),
                  constant_values=N_PLAYLIST).reshape(-1, GPI)
    g_t = jnp.pad(edge_dst, (0, ep)).reshape(-1, GPI)
    s_t = jnp.pad(edge_src, (0, ep), constant_values=N_TRACK).reshape(-1, GPI)
    lp = LPAD - E_LBL
    ls = jnp.pad(label_src, (0, lp))
    ld = jnp.pad(label_dst, (0, lp))
    zblk = jnp.zeros((128, D), jnp.float32)
    zblkq = jnp.zeros((512, DQ), jnp.float32)

    xt = _lin1(x_track, W_lt, b_lt)
    xp = _lin1(x_playlist, W_lp, b_lp)

    for (Wrc, Wqc, bc, Wrr, Wqr, br) in (
            (Wrel_c0, Wroot_c0, b_c0, Wrel_r0, Wroot_r0, b_r0),
            (Wrel_c1, Wroot_c1, b_c1, Wrel_r1, Wroot_r1, b_r1)):
        xq = jnp.concatenate(
            [xp[:, DQ * q:DQ * (q + 1)] for q in range(4)], axis=0)
        agg_p2 = _seg_partial_call(xt, g_p, s_p, zblk)
        agg_tq = _seg_quarters_call(xq, g_t, s_t, zblkq)
        aggq = agg_tq.reshape(4, QACC, DQ)[:, :N_TRACK, :]
        xp_new = _comb3(agg_p2[:N_PLAYLIST], agg_p2[N_PLAYLIST:], xp,
                        Wrc, Wqc, bc)
        xt_new = _comb2q(aggq, xt, Wrr.reshape(4, DQ, D), Wqr, br)
        xt, xp = xt_new, xp_new

    pred = _pred_call(xt, xp, ls, ld)
    return pred[:E_LBL]


# speedup vs baseline: 2.2379x; 1.0261x over previous
import functools

import jax
import jax.numpy as jnp
from jax import lax
from jax.experimental import pallas as pl
from jax.experimental.pallas import tpu as pltpu
from jax.experimental.pallas import tpu_sc as plsc

N_TRACK = 50000
N_PLAYLIST = 10000
E = 600000
E_LBL = 100000
D = 128
DQ = 32

CHUNK = 1024
GPI = 128
EPAD = 622592
CHUNKS_A = 19
CHUNKS_B = 38
LPAD = 102400

ACC_P = 10240
QACC = 50048


def _zero_acc(zblk, rbuf, acc, s, rows_per_sub, blk):
    pltpu.sync_copy(zblk, rbuf.at[pl.ds(0, blk)])
    n_full = rows_per_sub // blk
    rem = rows_per_sub - n_full * blk
    base = s * rows_per_sub
    for j in range(n_full):
        pltpu.sync_copy(rbuf.at[pl.ds(0, blk)],
                        acc.at[pl.ds(base + j * blk, blk)])
    if rem:
        pltpu.sync_copy(rbuf.at[pl.ds(0, rem)],
                        acc.at[pl.ds(base + n_full * blk, rem)])


def _gs_chunk(table, acc, gbuf, dbuf, rbuf, sem_g, sem_s, inflight):
    prev = [None, None]
    for h in range(8 // inflight):
        p = h % 2
        if prev[p] is not None:
            for hd in prev[p]:
                hd.wait()
        gs = []
        for j in range(inflight):
            gs.append(pltpu.async_copy(
                table.at[gbuf.at[h * inflight + j]],
                rbuf.at[pl.ds((p * inflight + j) * GPI, GPI)], sem_g))
        for hd in gs:
            hd.wait()
        ss = []
        for j in range(inflight):
            ss.append(pltpu.async_copy(
                rbuf.at[pl.ds((p * inflight + j) * GPI, GPI)],
                acc.at[dbuf.at[h * inflight + j]], sem_s, add=True))
        prev[p] = ss
    for p in range(2):
        if prev[p] is not None:
            for hd in prev[p]:
                hd.wait()


def _seg_partial_call(table, gidx2, didx2, zblk):
    mesh = plsc.VectorSubcoreMesh(core_axis_name="c", subcore_axis_name="s")

    @functools.partial(
        pl.kernel, mesh=mesh,
        out_type=jax.ShapeDtypeStruct((2 * N_PLAYLIST, D), jnp.float32),
        scratch_types=[
            pltpu.VMEM((8, GPI), jnp.int32),
            pltpu.VMEM((8, GPI), jnp.int32),
            pltpu.VMEM((256, D), jnp.float32),
            pltpu.VMEM_SHARED((ACC_P, D), jnp.float32),
            pltpu.SemaphoreType.DMA,
            pltpu.SemaphoreType.DMA,
        ],
    )
    def k(table_h, gidx_h, didx_h, zblk_h, out_h,
          gbuf, dbuf, rbuf, acc, sem_g, sem_s):
        c = lax.axis_index("c")
        s = lax.axis_index("s")
        _zero_acc(zblk_h, rbuf, acc, s, ACC_P // 16, 128)
        plsc.subcore_barrier()

        first = (c * 16 + s) * CHUNKS_A

        def body(i, carry):
            row0 = (first + i) * 8
            pltpu.sync_copy(gidx_h.at[pl.ds(row0, 8)], gbuf)
            pltpu.sync_copy(didx_h.at[pl.ds(row0, 8)], dbuf)
            _gs_chunk(table_h, acc, gbuf, dbuf, rbuf, sem_g, sem_s, 1)
            return carry

        lax.fori_loop(0, CHUNKS_A, body, 0)
        plsc.subcore_barrier()

        for j in range(8):
            cid = j * 16 + s
            @pl.when(cid < 125)
            def _copy():
                r0 = cid * 80
                pltpu.sync_copy(acc.at[pl.ds(r0, 80)], rbuf.at[pl.ds(0, 80)])
                pltpu.sync_copy(rbuf.at[pl.ds(0, 80)],
                                out_h.at[pl.ds(c * N_PLAYLIST + r0, 80)])

    return k(table, gidx2, didx2, zblk)


def _seg_quarters_call(tableq, gidx2, didx2, zblkq):
    mesh = plsc.VectorSubcoreMesh(core_axis_name="c", subcore_axis_name="s")

    @functools.partial(
        pl.kernel, mesh=mesh,
        compiler_params=pltpu.CompilerParams(use_tc_tiling_on_sc=False),
        out_type=jax.ShapeDtypeStruct((4 * QACC, DQ), jnp.float32),
        scratch_types=[
            pltpu.VMEM((8, GPI), jnp.int32),
            pltpu.VMEM((8, GPI), jnp.int32),
            pltpu.VMEM((512, DQ), jnp.float32),
            pltpu.VMEM_SHARED((QACC, DQ), jnp.float32),
            pltpu.SemaphoreType.DMA,
            pltpu.SemaphoreType.DMA,
        ],
    )
    def k(table_h, gidx_h, didx_h, zblk_h, out_h,
          gbuf, dbuf, rbuf, acc, sem_g, sem_s):
        c = lax.axis_index("c")
        s = lax.axis_index("s")
        rows_per_sub = QACC // 16

        for b in range(2):
            q = 2 * c + b
            off = q * N_PLAYLIST

            _zero_acc(zblk_h, rbuf, acc, s, rows_per_sub, 512)
            plsc.subcore_barrier()

            def body(i, carry):
                row0 = (s * CHUNKS_B + i) * 8
                pltpu.sync_copy(gidx_h.at[pl.ds(row0, 8)], gbuf)
                pltpu.sync_copy(didx_h.at[pl.ds(row0, 8)], dbuf)
                for j in range(8):
                    for kk in range(8):
                        v = gbuf[j, pl.ds(kk * 16, 16)]
                        gbuf[j, pl.ds(kk * 16, 16)] = v + off
                _gs_chunk(table_h, acc, gbuf, dbuf, rbuf, sem_g, sem_s, 2)
                return carry

            lax.fori_loop(0, CHUNKS_B, body, 0)
            plsc.subcore_barrier()

            base = s * rows_per_sub
            for j in range(6):
                pltpu.sync_copy(acc.at[pl.ds(base + j * 512, 512)], rbuf)
                pltpu.sync_copy(
                    rbuf, out_h.at[pl.ds(q * QACC + base + j * 512, 512)])
            pltpu.sync_copy(acc.at[pl.ds(base + 3072, 56)],
                            rbuf.at[pl.ds(0, 56)])
            pltpu.sync_copy(rbuf.at[pl.ds(0, 56)],
                            out_h.at[pl.ds(q * QACC + base + 3072, 56)])
            plsc.subcore_barrier()

    return k(tableq, gidx2, didx2, zblkq)


def _pred_call(xt, xp, lsrc, ldst):
    mesh = plsc.VectorSubcoreMesh(core_axis_name="c", subcore_axis_name="s")

    @functools.partial(
        pl.kernel, mesh=mesh,
        compiler_params=pltpu.CompilerParams(needs_layout_passes=False),
        out_type=jax.ShapeDtypeStruct((LPAD,), jnp.float32),
        scratch_types=[
            pltpu.VMEM((GPI,), jnp.int32),
            pltpu.VMEM((GPI,), jnp.int32),
            pltpu.VMEM((GPI, D), jnp.float32),
            pltpu.VMEM((GPI, D), jnp.float32),
            pltpu.VMEM((GPI,), jnp.float32),
            pltpu.SemaphoreType.DMA,
        ],
    )
    def k(xt_h, xp_h, ls_h, ld_h, out_h, sbuf, dbuf, ra, rb, obuf, sem):
        c = lax.axis_index("c")
        s = lax.axis_index("s")
        wid = c * 16 + s

        def body(i, carry):
            row = wid * 25 + i
            pltpu.sync_copy(ls_h.at[pl.ds(row * GPI, GPI)], sbuf)
            pltpu.sync_copy(ld_h.at[pl.ds(row * GPI, GPI)], dbuf)
            ha = pltpu.make_async_copy(xt_h.at[sbuf], ra, sem)
            hb = pltpu.make_async_copy(xp_h.at[dbuf], rb, sem)
            ha.start(); hb.start()
            ha.wait(); hb.wait()

            lane = lax.iota(jnp.int32, 16)

            def dot_group(g, carry2):
                accvec = jnp.zeros((16,), jnp.float32)
                for rr in range(16):
                    r = g * 16 + rr
                    acc = ra[r, pl.ds(0, 16)] * rb[r, pl.ds(0, 16)]
                    for kk in range(1, 8):
                        acc = acc + (ra[r, pl.ds(kk * 16, 16)]
                                     * rb[r, pl.ds(kk * 16, 16)])
                    accvec = jnp.where(lane == rr, jnp.sum(acc), accvec)
                obuf[pl.ds(g * 16, 16)] = accvec
                return carry2

            lax.fori_loop(0, 8, dot_group, 0)
            pltpu.sync_copy(obuf, out_h.at[pl.ds(row * GPI, GPI)])
            return carry

        lax.fori_loop(0, 25, body, 0)

    return k(xt, xp, lsrc, ldst)


_RB = 2000


def _lin1(x, w, b):
    n = x.shape[0]

    def body(x_r, w_r, b_r, o_r):
        o_r[...] = jax.lax.dot_general(
            x_r[...], w_r[...], (((1,), (0,)), ((), ())),
            preferred_element_type=jnp.float32) + b_r[...]

    return pl.pallas_call(
        body,
        grid=(n // _RB,),
        in_specs=[
            pl.BlockSpec((_RB, D), lambda i: (i, 0)),
            pl.BlockSpec((D, D), lambda i: (0, 0)),
            pl.BlockSpec((1, D), lambda i: (0, 0)),
        ],
        out_specs=pl.BlockSpec((_RB, D), lambda i: (i, 0)),
        out_shape=jax.ShapeDtypeStruct((n, D), jnp.float32),
    )(x, w, b.reshape(1, D))


def _comb2q(aggq, x, w4, w2, b):
    n = x.shape[0]

    def body(a_r, x_r, w4_r, w2_r, b_r, o_r):
        mm = jax.lax.dot_general(
            x_r[...], w2_r[...], (((1,), (0,)), ((), ())),
            preferred_element_type=jnp.float32)
        for q in range(4):
            mm = mm + jax.lax.dot_general(
                a_r[q], w4_r[q], (((1,), (0,)), ((), ())),
                preferred_element_type=jnp.float32)
        o_r[...] = mm + b_r[...]

    return pl.pallas_call(
        body,
        grid=(n // _RB,),
        in_specs=[
            pl.BlockSpec((4, _RB, DQ), lambda i: (0, i, 0)),
            pl.BlockSpec((_RB, D), lambda i: (i, 0)),
            pl.BlockSpec((4, DQ, D), lambda i: (0, 0, 0)),
            pl.BlockSpec((D, D), lambda i: (0, 0)),
            pl.BlockSpec((1, D), lambda i: (0, 0)),
        ],
        out_specs=pl.BlockSpec((_RB, D), lambda i: (i, 0)),
        out_shape=jax.ShapeDtypeStruct((n, D), jnp.float32),
    )(aggq, x, w4, w2, b.reshape(1, D))


def _comb3(p0, p1, x, w1, w2, b):
    n = x.shape[0]

    def body(p0_r, p1_r, x_r, w1_r, w2_r, b_r, o_r):
        a = p0_r[...] + p1_r[...]
        mm = jax.lax.dot_general(
            a, w1_r[...], (((1,), (0,)), ((), ())),
            preferred_element_type=jnp.float32)
        mm = mm + jax.lax.dot_general(
            x_r[...], w2_r[...], (((1,), (0,)), ((), ())),
            preferred_element_type=jnp.float32)
        o_r[...] = mm + b_r[...]

    return pl.pallas_call(
        body,
        grid=(n // _RB,),
        in_specs=[
            pl.BlockSpec((_RB, D), lambda i: (i, 0)),
            pl.BlockSpec((_RB, D), lambda i: (i, 0)),
            pl.BlockSpec((_RB, D), lambda i: (i, 0)),
            pl.BlockSpec((D, D), lambda i: (0, 0)),
            pl.BlockSpec((D, D), lambda i: (0, 0)),
            pl.BlockSpec((1, D), lambda i: (0, 0)),
        ],
        out_specs=pl.BlockSpec((_RB, D), lambda i: (i, 0)),
        out_shape=jax.ShapeDtypeStruct((n, D), jnp.float32),
    )(p0, p1, x, w1, w2, b.reshape(1, D))


def kernel(x_track, x_playlist, edge_src, edge_dst, label_src, label_dst,
           W_lt, b_lt, W_lp, b_lp,
           Wrel_c0, Wroot_c0, b_c0, Wrel_r0, Wroot_r0, b_r0,
           Wrel_c1, Wroot_c1, b_c1, Wrel_r1, Wroot_r1, b_r1):
    ep = EPAD - E
    g_p = jnp.pad(edge_src, (0, ep)).reshape(-1, GPI)
    s_p = jnp.pad(edge_dst, (0, ep),
                  constant_values=N_PLAYLIST).reshape(-1, GPI)
    g_t = jnp.pad(edge_dst, (0, ep)).reshape(-1, GPI)
    s_t = jnp.pad(edge_src, (0, ep), constant_values=N_TRACK).reshape(-1, GPI)
    lp = LPAD - E_LBL
    ls = jnp.pad(label_src, (0, lp))
    ld = jnp.pad(label_dst, (0, lp))
    zblk = jnp.zeros((128, D), jnp.float32)
    zblkq = jnp.zeros((512, DQ), jnp.float32)

    xt = _lin1(x_track, W_lt, b_lt)
    xp = _lin1(x_playlist, W_lp, b_lp)

    for (Wrc, Wqc, bc, Wrr, Wqr, br) in (
            (Wrel_c0, Wroot_c0, b_c0, Wrel_r0, Wroot_r0, b_r0),
            (Wrel_c1, Wroot_c1, b_c1, Wrel_r1, Wroot_r1, b_r1)):
        xq = jnp.concatenate(
            [xp[:, DQ * q:DQ * (q + 1)] for q in range(4)], axis=0)
        agg_p2 = _seg_partial_call(xt, g_p, s_p, zblk)
        agg_tq = _seg_quarters_call(xq, g_t, s_t, zblkq)
        aggq = agg_tq.reshape(4, QACC, DQ)[:, :N_TRACK, :]
        xp_new = _comb3(agg_p2[:N_PLAYLIST], agg_p2[N_PLAYLIST:], xp,
                        Wrc, Wqc, bc)
        xt_new = _comb2q(aggq, xt, Wrr.reshape(4, DQ, D), Wqr, br)
        xt, xp = xt_new, xp_new

    pred = _pred_call(xt, xp, ls, ld)
    return pred[:E_LBL]


# speedup vs baseline: 2.9155x; 1.3028x over previous
import functools

import jax
import jax.numpy as jnp
from jax import lax
from jax.experimental import pallas as pl
from jax.experimental.pallas import tpu as pltpu
from jax.experimental.pallas import tpu_sc as plsc

N_TRACK = 50000
N_PLAYLIST = 10000
E = 600000
E_LBL = 100000
D = 128
DQ = 32

CHUNK = 1024
GPI = 128
EPAD = 622592
CHUNKS_A = 19
CHUNKS_B = 38
LPAD = 102400

ACC_P = 10240
QACC = 50048


def _zero_acc(zblk, rbuf, acc, s, rows_per_sub, blk):
    pltpu.sync_copy(zblk, rbuf.at[pl.ds(0, blk)])
    n_full = rows_per_sub // blk
    rem = rows_per_sub - n_full * blk
    base = s * rows_per_sub
    for j in range(n_full):
        pltpu.sync_copy(rbuf.at[pl.ds(0, blk)],
                        acc.at[pl.ds(base + j * blk, blk)])
    if rem:
        pltpu.sync_copy(rbuf.at[pl.ds(0, rem)],
                        acc.at[pl.ds(base + n_full * blk, rem)])


def _gs_chunk(table, acc, gbuf, dbuf, rbuf, sem_g, sem_s, inflight):
    prev = [None, None]
    for h in range(8 // inflight):
        p = h % 2
        if prev[p] is not None:
            for hd in prev[p]:
                hd.wait()
        gs = []
        for j in range(inflight):
            gs.append(pltpu.async_copy(
                table.at[gbuf.at[h * inflight + j]],
                rbuf.at[pl.ds((p * inflight + j) * GPI, GPI)], sem_g))
        for hd in gs:
            hd.wait()
        ss = []
        for j in range(inflight):
            ss.append(pltpu.async_copy(
                rbuf.at[pl.ds((p * inflight + j) * GPI, GPI)],
                acc.at[dbuf.at[h * inflight + j]], sem_s, add=True))
        prev[p] = ss
    for p in range(2):
        if prev[p] is not None:
            for hd in prev[p]:
                hd.wait()


def _seg_partial_call(table, gidx2, didx2, zblk):
    mesh = plsc.VectorSubcoreMesh(core_axis_name="c", subcore_axis_name="s")

    @functools.partial(
        pl.kernel, mesh=mesh,
        out_type=jax.ShapeDtypeStruct((2 * N_PLAYLIST, D), jnp.float32),
        scratch_types=[
            pltpu.VMEM((8, GPI), jnp.int32),
            pltpu.VMEM((8, GPI), jnp.int32),
            pltpu.VMEM((256, D), jnp.float32),
            pltpu.VMEM_SHARED((ACC_P, D), jnp.float32),
            pltpu.SemaphoreType.DMA,
            pltpu.SemaphoreType.DMA,
        ],
    )
    def k(table_h, gidx_h, didx_h, zblk_h, out_h,
          gbuf, dbuf, rbuf, acc, sem_g, sem_s):
        c = lax.axis_index("c")
        s = lax.axis_index("s")
        _zero_acc(zblk_h, rbuf, acc, s, ACC_P // 16, 128)
        plsc.subcore_barrier()

        first = (c * 16 + s) * CHUNKS_A

        def body(i, carry):
            row0 = (first + i) * 8
            pltpu.sync_copy(gidx_h.at[pl.ds(row0, 8)], gbuf)
            pltpu.sync_copy(didx_h.at[pl.ds(row0, 8)], dbuf)
            _gs_chunk(table_h, acc, gbuf, dbuf, rbuf, sem_g, sem_s, 1)
            return carry

        lax.fori_loop(0, CHUNKS_A, body, 0)
        plsc.subcore_barrier()

        for j in range(8):
            cid = j * 16 + s
            @pl.when(cid < 125)
            def _copy():
                r0 = cid * 80
                pltpu.sync_copy(acc.at[pl.ds(r0, 80)], rbuf.at[pl.ds(0, 80)])
                pltpu.sync_copy(rbuf.at[pl.ds(0, 80)],
                                out_h.at[pl.ds(c * N_PLAYLIST + r0, 80)])

    return k(table, gidx2, didx2, zblk)


def _seg_quarters_call(tableq, gidx2, didx2, zblkq):
    mesh = plsc.VectorSubcoreMesh(core_axis_name="c", subcore_axis_name="s")

    @functools.partial(
        pl.kernel, mesh=mesh,
        compiler_params=pltpu.CompilerParams(use_tc_tiling_on_sc=False),
        out_type=jax.ShapeDtypeStruct((4 * QACC, DQ), jnp.float32),
        scratch_types=[
            pltpu.VMEM((8, GPI), jnp.int32),
            pltpu.VMEM((8, GPI), jnp.int32),
            pltpu.VMEM((256, DQ), jnp.float32),
            pltpu.VMEM_SHARED((N_PLAYLIST, DQ), jnp.float32),
            pltpu.VMEM_SHARED((QACC, DQ), jnp.float32),
            pltpu.SemaphoreType.DMA,
            pltpu.SemaphoreType.DMA,
        ],
    )
    def k(table_h, gidx_h, didx_h, zblk_h, out_h,
          gbuf, dbuf, rbuf, table_s, acc, sem_g, sem_s):
        c = lax.axis_index("c")
        s = lax.axis_index("s")
        rows_per_sub = QACC // 16

        for b in range(2):
            q = 2 * c + b

            @pl.when(s < 15)
            def _stage():
                pltpu.sync_copy(table_h.at[pl.ds(q * N_PLAYLIST + s * 640, 640)],
                                table_s.at[pl.ds(s * 640, 640)])
            @pl.when(s == 15)
            def _stage_tail():
                pltpu.sync_copy(table_h.at[pl.ds(q * N_PLAYLIST + 9600, 400)],
                                table_s.at[pl.ds(9600, 400)])
            _zero_acc(zblk_h, rbuf, acc, s, rows_per_sub, 256)
            plsc.subcore_barrier()

            def body(i, carry):
                row0 = (s * CHUNKS_B + i) * 8
                pltpu.sync_copy(gidx_h.at[pl.ds(row0, 8)], gbuf)
                pltpu.sync_copy(didx_h.at[pl.ds(row0, 8)], dbuf)
                _gs_chunk(table_s, acc, gbuf, dbuf, rbuf, sem_g, sem_s, 1)
                return carry

            lax.fori_loop(0, CHUNKS_B, body, 0)
            plsc.subcore_barrier()

            base = s * rows_per_sub
            for j in range(12):
                pltpu.sync_copy(acc.at[pl.ds(base + j * 256, 256)], rbuf)
                pltpu.sync_copy(
                    rbuf, out_h.at[pl.ds(q * QACC + base + j * 256, 256)])
            pltpu.sync_copy(acc.at[pl.ds(base + 3072, 56)],
                            rbuf.at[pl.ds(0, 56)])
            pltpu.sync_copy(rbuf.at[pl.ds(0, 56)],
                            out_h.at[pl.ds(q * QACC + base + 3072, 56)])
            plsc.subcore_barrier()

    return k(tableq, gidx2, didx2, zblkq)


def _pred_call(xt, xp, lsrc, ldst):
    mesh = plsc.VectorSubcoreMesh(core_axis_name="c", subcore_axis_name="s")

    @functools.partial(
        pl.kernel, mesh=mesh,
        compiler_params=pltpu.CompilerParams(needs_layout_passes=False),
        out_type=jax.ShapeDtypeStruct((LPAD,), jnp.float32),
        scratch_types=[
            pltpu.VMEM((GPI,), jnp.int32),
            pltpu.VMEM((GPI,), jnp.int32),
            pltpu.VMEM((GPI, D), jnp.float32),
            pltpu.VMEM((GPI, D), jnp.float32),
            pltpu.VMEM((GPI,), jnp.float32),
            pltpu.SemaphoreType.DMA,
        ],
    )
    def k(xt_h, xp_h, ls_h, ld_h, out_h, sbuf, dbuf, ra, rb, obuf, sem):
        c = lax.axis_index("c")
        s = lax.axis_index("s")
        wid = c * 16 + s

        def body(i, carry):
            row = wid * 25 + i
            pltpu.sync_copy(ls_h.at[pl.ds(row * GPI, GPI)], sbuf)
            pltpu.sync_copy(ld_h.at[pl.ds(row * GPI, GPI)], dbuf)
            ha = pltpu.make_async_copy(xt_h.at[sbuf], ra, sem)
            hb = pltpu.make_async_copy(xp_h.at[dbuf], rb, sem)
            ha.start(); hb.start()
            ha.wait(); hb.wait()

            lane = lax.iota(jnp.int32, 16)

            def dot_group(g, carry2):
                accvec = jnp.zeros((16,), jnp.float32)
                for rr in range(16):
                    r = g * 16 + rr
                    acc = ra[r, pl.ds(0, 16)] * rb[r, pl.ds(0, 16)]
                    for kk in range(1, 8):
                        acc = acc + (ra[r, pl.ds(kk * 16, 16)]
                                     * rb[r, pl.ds(kk * 16, 16)])
                    accvec = jnp.where(lane == rr, jnp.sum(acc), accvec)
                obuf[pl.ds(g * 16, 16)] = accvec
                return carry2

            lax.fori_loop(0, 8, dot_group, 0)
            pltpu.sync_copy(obuf, out_h.at[pl.ds(row * GPI, GPI)])
            return carry

        lax.fori_loop(0, 25, body, 0)

    return k(xt, xp, lsrc, ldst)


_RB = 2000


def _lin1(x, w, b):
    n = x.shape[0]

    def body(x_r, w_r, b_r, o_r):
        o_r[...] = jax.lax.dot_general(
            x_r[...], w_r[...], (((1,), (0,)), ((), ())),
            preferred_element_type=jnp.float32) + b_r[...]

    return pl.pallas_call(
        body,
        grid=(n // _RB,),
        in_specs=[
            pl.BlockSpec((_RB, D), lambda i: (i, 0)),
            pl.BlockSpec((D, D), lambda i: (0, 0)),
            pl.BlockSpec((1, D), lambda i: (0, 0)),
        ],
        out_specs=pl.BlockSpec((_RB, D), lambda i: (i, 0)),
        out_shape=jax.ShapeDtypeStruct((n, D), jnp.float32),
    )(x, w, b.reshape(1, D))


def _comb2q(aggq, x, w4, w2, b):
    n = x.shape[0]

    def body(a_r, x_r, w4_r, w2_r, b_r, o_r):
        mm = jax.lax.dot_general(
            x_r[...], w2_r[...], (((1,), (0,)), ((), ())),
            preferred_element_type=jnp.float32)
        for q in range(4):
            mm = mm + jax.lax.dot_general(
                a_r[q], w4_r[q], (((1,), (0,)), ((), ())),
                preferred_element_type=jnp.float32)
        o_r[...] = mm + b_r[...]

    return pl.pallas_call(
        body,
        grid=(n // _RB,),
        in_specs=[
            pl.BlockSpec((4, _RB, DQ), lambda i: (0, i, 0)),
            pl.BlockSpec((_RB, D), lambda i: (i, 0)),
            pl.BlockSpec((4, DQ, D), lambda i: (0, 0, 0)),
            pl.BlockSpec((D, D), lambda i: (0, 0)),
            pl.BlockSpec((1, D), lambda i: (0, 0)),
        ],
        out_specs=pl.BlockSpec((_RB, D), lambda i: (i, 0)),
        out_shape=jax.ShapeDtypeStruct((n, D), jnp.float32),
    )(aggq, x, w4, w2, b.reshape(1, D))


def _comb3(p0, p1, x, w1, w2, b):
    n = x.shape[0]

    def body(p0_r, p1_r, x_r, w1_r, w2_r, b_r, o_r):
        a = p0_r[...] + p1_r[...]
        mm = jax.lax.dot_general(
            a, w1_r[...], (((1,), (0,)), ((), ())),
            preferred_element_type=jnp.float32)
        mm = mm + jax.lax.dot_general(
            x_r[...], w2_r[...], (((1,), (0,)), ((), ())),
            preferred_element_type=jnp.float32)
        o_r[...] = mm + b_r[...]

    return pl.pallas_call(
        body,
        grid=(n // _RB,),
        in_specs=[
            pl.BlockSpec((_RB, D), lambda i: (i, 0)),
            pl.BlockSpec((_RB, D), lambda i: (i, 0)),
            pl.BlockSpec((_RB, D), lambda i: (i, 0)),
            pl.BlockSpec((D, D), lambda i: (0, 0)),
            pl.BlockSpec((D, D), lambda i: (0, 0)),
            pl.BlockSpec((1, D), lambda i: (0, 0)),
        ],
        out_specs=pl.BlockSpec((_RB, D), lambda i: (i, 0)),
        out_shape=jax.ShapeDtypeStruct((n, D), jnp.float32),
    )(p0, p1, x, w1, w2, b.reshape(1, D))


def kernel(x_track, x_playlist, edge_src, edge_dst, label_src, label_dst,
           W_lt, b_lt, W_lp, b_lp,
           Wrel_c0, Wroot_c0, b_c0, Wrel_r0, Wroot_r0, b_r0,
           Wrel_c1, Wroot_c1, b_c1, Wrel_r1, Wroot_r1, b_r1):
    ep = EPAD - E
    g_p = jnp.pad(edge_src, (0, ep)).reshape(-1, GPI)
    s_p = jnp.pad(edge_dst, (0, ep),
                  constant_values=N_PLAYLIST).reshape(-1, GPI)
    g_t = jnp.pad(edge_dst, (0, ep)).reshape(-1, GPI)
    s_t = jnp.pad(edge_src, (0, ep), constant_values=N_TRACK).reshape(-1, GPI)
    lp = LPAD - E_LBL
    ls = jnp.pad(label_src, (0, lp))
    ld = jnp.pad(label_dst, (0, lp))
    zblk = jnp.zeros((128, D), jnp.float32)
    zblkq = jnp.zeros((256, DQ), jnp.float32)

    xt = _lin1(x_track, W_lt, b_lt)
    xp = _lin1(x_playlist, W_lp, b_lp)

    for (Wrc, Wqc, bc, Wrr, Wqr, br) in (
            (Wrel_c0, Wroot_c0, b_c0, Wrel_r0, Wroot_r0, b_r0),
            (Wrel_c1, Wroot_c1, b_c1, Wrel_r1, Wroot_r1, b_r1)):
        xq = jnp.concatenate(
            [xp[:, DQ * q:DQ * (q + 1)] for q in range(4)], axis=0)
        agg_p2 = _seg_partial_call(xt, g_p, s_p, zblk)
        agg_tq = _seg_quarters_call(xq, g_t, s_t, zblkq)
        aggq = agg_tq.reshape(4, QACC, DQ)[:, :N_TRACK, :]
        xp_new = _comb3(agg_p2[:N_PLAYLIST], agg_p2[N_PLAYLIST:], xp,
                        Wrc, Wqc, bc)
        xt_new = _comb2q(aggq, xt, Wrr.reshape(4, DQ, D), Wqr, br)
        xt, xp = xt_new, xp_new

    pred = _pred_call(xt, xp, ls, ld)
    return pred[:E_LBL]


# speedup vs baseline: 5.0851x; 1.7442x over previous
import functools

import jax
import jax.numpy as jnp
from jax import lax
from jax.experimental import pallas as pl
from jax.experimental.pallas import tpu as pltpu
from jax.experimental.pallas import tpu_sc as plsc

N_TRACK = 50000
N_PLAYLIST = 10000
E = 600000
E_LBL = 100000
D = 128
DQ = 32

CHUNK = 1024
GPI = 128
EPAD = 622592
CHUNKS_B = 38
LPAD = 102400

QACC = 50048
PACC = 10240


def _zero_acc(zblk, rbuf, acc, s, rows_per_sub, blk):
    pltpu.sync_copy(zblk, rbuf.at[pl.ds(0, blk)])
    n_full = rows_per_sub // blk
    rem = rows_per_sub - n_full * blk
    base = s * rows_per_sub
    for j in range(n_full):
        pltpu.sync_copy(rbuf.at[pl.ds(0, blk)],
                        acc.at[pl.ds(base + j * blk, blk)])
    if rem:
        pltpu.sync_copy(rbuf.at[pl.ds(0, rem)],
                        acc.at[pl.ds(base + n_full * blk, rem)])


def _gs_chunk(table, acc, gbuf, dbuf, rbuf, sem_g, sem_s):
    prev = [None, None]
    for h in range(8):
        p = h % 2
        if prev[p] is not None:
            prev[p].wait()
        g = pltpu.async_copy(table.at[gbuf.at[h]],
                             rbuf.at[pl.ds(p * GPI, GPI)], sem_g)
        g.wait()
        prev[p] = pltpu.async_copy(rbuf.at[pl.ds(p * GPI, GPI)],
                                   acc.at[dbuf.at[h]], sem_s, add=True)
    for p in range(2):
        if prev[p] is not None:
            prev[p].wait()


def _seg_q_call(tableq, gidx2, didx2, zblkq, n_table, n_acc):
    mesh = plsc.VectorSubcoreMesh(core_axis_name="c", subcore_axis_name="s")

    st = (n_table // 16 + 7) // 8 * 8
    st_last = n_table - 15 * st
    rows_per_sub = n_acc // 16

    @functools.partial(
        pl.kernel, mesh=mesh,
        compiler_params=pltpu.CompilerParams(use_tc_tiling_on_sc=False),
        out_type=jax.ShapeDtypeStruct((4 * n_acc, DQ), jnp.float32),
        scratch_types=[
            pltpu.VMEM((8, GPI), jnp.int32),
            pltpu.VMEM((8, GPI), jnp.int32),
            pltpu.VMEM((256, DQ), jnp.float32),
            pltpu.VMEM_SHARED((n_table, DQ), jnp.float32),
            pltpu.VMEM_SHARED((n_acc, DQ), jnp.float32),
            pltpu.SemaphoreType.DMA,
            pltpu.SemaphoreType.DMA,
        ],
    )
    def k(table_h, gidx_h, didx_h, zblk_h, out_h,
          gbuf, dbuf, rbuf, table_s, acc, sem_g, sem_s):
        c = lax.axis_index("c")
        s = lax.axis_index("s")

        for b in range(2):
            q = 2 * c + b

            @pl.when(s < 15)
            def _stage():
                pltpu.sync_copy(
                    table_h.at[pl.ds(q * n_table + s * st, st)],
                    table_s.at[pl.ds(s * st, st)])
            @pl.when(s == 15)
            def _stage_tail():
                pltpu.sync_copy(
                    table_h.at[pl.ds(q * n_table + 15 * st, st_last)],
                    table_s.at[pl.ds(15 * st, st_last)])
            _zero_acc(zblk_h, rbuf, acc, s, rows_per_sub, 256)
            plsc.subcore_barrier()

            def body(i, carry):
                row0 = (s * CHUNKS_B + i) * 8
                pltpu.sync_copy(gidx_h.at[pl.ds(row0, 8)], gbuf)
                pltpu.sync_copy(didx_h.at[pl.ds(row0, 8)], dbuf)
                _gs_chunk(table_s, acc, gbuf, dbuf, rbuf, sem_g, sem_s)
                return carry

            lax.fori_loop(0, CHUNKS_B, body, 0)
            plsc.subcore_barrier()

            base = s * rows_per_sub
            n_full = rows_per_sub // 256
            rem = rows_per_sub - n_full * 256
            for j in range(n_full):
                pltpu.sync_copy(acc.at[pl.ds(base + j * 256, 256)], rbuf)
                pltpu.sync_copy(
                    rbuf, out_h.at[pl.ds(q * n_acc + base + j * 256, 256)])
            if rem:
                pltpu.sync_copy(acc.at[pl.ds(base + n_full * 256, rem)],
                                rbuf.at[pl.ds(0, rem)])
                pltpu.sync_copy(
                    rbuf.at[pl.ds(0, rem)],
                    out_h.at[pl.ds(q * n_acc + base + n_full * 256, rem)])
            plsc.subcore_barrier()

    return k(tableq, gidx2, didx2, zblkq)


def _pred_call(xt, xp, lsrc, ldst):
    mesh = plsc.VectorSubcoreMesh(core_axis_name="c", subcore_axis_name="s")

    @functools.partial(
        pl.kernel, mesh=mesh,
        compiler_params=pltpu.CompilerParams(needs_layout_passes=False),
        out_type=jax.ShapeDtypeStruct((LPAD,), jnp.float32),
        scratch_types=[
            pltpu.VMEM((GPI,), jnp.int32),
            pltpu.VMEM((GPI,), jnp.int32),
            pltpu.VMEM((GPI, D), jnp.float32),
            pltpu.VMEM((GPI, D), jnp.float32),
            pltpu.VMEM((GPI,), jnp.float32),
            pltpu.SemaphoreType.DMA,
        ],
    )
    def k(xt_h, xp_h, ls_h, ld_h, out_h, sbuf, dbuf, ra, rb, obuf, sem):
        c = lax.axis_index("c")
        s = lax.axis_index("s")
        wid = c * 16 + s

        def body(i, carry):
            row = wid * 25 + i
            pltpu.sync_copy(ls_h.at[pl.ds(row * GPI, GPI)], sbuf)
            pltpu.sync_copy(ld_h.at[pl.ds(row * GPI, GPI)], dbuf)
            ha = pltpu.make_async_copy(xt_h.at[sbuf], ra, sem)
            hb = pltpu.make_async_copy(xp_h.at[dbuf], rb, sem)
            ha.start(); hb.start()
            ha.wait(); hb.wait()

            lane = lax.iota(jnp.int32, 16)

            def dot_group(g, carry2):
                accvec = jnp.zeros((16,), jnp.float32)
                for rr in range(16):
                    r = g * 16 + rr
                    acc = ra[r, pl.ds(0, 16)] * rb[r, pl.ds(0, 16)]
                    for kk in range(1, 8):
                        acc = acc + (ra[r, pl.ds(kk * 16, 16)]
                                     * rb[r, pl.ds(kk * 16, 16)])
                    accvec = jnp.where(lane == rr, jnp.sum(acc), accvec)
                obuf[pl.ds(g * 16, 16)] = accvec
                return carry2

            lax.fori_loop(0, 8, dot_group, 0)
            pltpu.sync_copy(obuf, out_h.at[pl.ds(row * GPI, GPI)])
            return carry

        lax.fori_loop(0, 25, body, 0)

    return k(xt, xp, lsrc, ldst)


_RB = 2000


def _lin1(x, w, b):
    n = x.shape[0]

    def body(x_r, w_r, b_r, o_r):
        o_r[...] = jax.lax.dot_general(
            x_r[...], w_r[...], (((1,), (0,)), ((), ())),
            preferred_element_type=jnp.float32) + b_r[...]

    return pl.pallas_call(
        body,
        grid=(n // _RB,),
        in_specs=[
            pl.BlockSpec((_RB, D), lambda i: (i, 0)),
            pl.BlockSpec((D, D), lambda i: (0, 0)),
            pl.BlockSpec((1, D), lambda i: (0, 0)),
        ],
        out_specs=pl.BlockSpec((_RB, D), lambda i: (i, 0)),
        out_shape=jax.ShapeDtypeStruct((n, D), jnp.float32),
    )(x, w, b.reshape(1, D))


def _comb2q(aggq, x, w4, w2, b):
    n = x.shape[0]

    def body(a_r, x_r, w4_r, w2_r, b_r, o_r):
        mm = jax.lax.dot_general(
            x_r[...], w2_r[...], (((1,), (0,)), ((), ())),
            preferred_element_type=jnp.float32)
        for q in range(4):
            mm = mm + jax.lax.dot_general(
                a_r[q], w4_r[q], (((1,), (0,)), ((), ())),
                preferred_element_type=jnp.float32)
        o_r[...] = mm + b_r[...]

    return pl.pallas_call(
        body,
        grid=(n // _RB,),
        in_specs=[
            pl.BlockSpec((4, _RB, DQ), lambda i: (0, i, 0)),
            pl.BlockSpec((_RB, D), lambda i: (i, 0)),
            pl.BlockSpec((4, DQ, D), lambda i: (0, 0, 0)),
            pl.BlockSpec((D, D), lambda i: (0, 0)),
            pl.BlockSpec((1, D), lambda i: (0, 0)),
        ],
        out_specs=pl.BlockSpec((_RB, D), lambda i: (i, 0)),
        out_shape=jax.ShapeDtypeStruct((n, D), jnp.float32),
    )(aggq, x, w4, w2, b.reshape(1, D))


def _quarters(x):
    return jnp.concatenate(
        [x[:, DQ * q:DQ * (q + 1)] for q in range(4)], axis=0)


def kernel(x_track, x_playlist, edge_src, edge_dst, label_src, label_dst,
           W_lt, b_lt, W_lp, b_lp,
           Wrel_c0, Wroot_c0, b_c0, Wrel_r0, Wroot_r0, b_r0,
           Wrel_c1, Wroot_c1, b_c1, Wrel_r1, Wroot_r1, b_r1):
    ep = EPAD - E
    g_p = jnp.pad(edge_src, (0, ep)).reshape(-1, GPI)
    s_p = jnp.pad(edge_dst, (0, ep),
                  constant_values=N_PLAYLIST).reshape(-1, GPI)
    g_t = jnp.pad(edge_dst, (0, ep)).reshape(-1, GPI)
    s_t = jnp.pad(edge_src, (0, ep), constant_values=N_TRACK).reshape(-1, GPI)
    lp = LPAD - E_LBL
    ls = jnp.pad(label_src, (0, lp))
    ld = jnp.pad(label_dst, (0, lp))
    zblkq = jnp.zeros((256, DQ), jnp.float32)

    xt = _lin1(x_track, W_lt, b_lt)
    xp = _lin1(x_playlist, W_lp, b_lp)

    for (Wrc, Wqc, bc, Wrr, Wqr, br) in (
            (Wrel_c0, Wroot_c0, b_c0, Wrel_r0, Wroot_r0, b_r0),
            (Wrel_c1, Wroot_c1, b_c1, Wrel_r1, Wroot_r1, b_r1)):
        agg_pq = _seg_q_call(_quarters(xt), g_p, s_p, zblkq, N_TRACK, PACC)
        agg_tq = _seg_q_call(_quarters(xp), g_t, s_t, zblkq, N_PLAYLIST, QACC)
        apq = agg_pq.reshape(4, PACC, DQ)[:, :N_PLAYLIST, :]
        atq = agg_tq.reshape(4, QACC, DQ)[:, :N_TRACK, :]
        xp_new = _comb2q(apq, xp, Wrc.reshape(4, DQ, D), Wqc, bc)
        xt_new = _comb2q(atq, xt, Wrr.reshape(4, DQ, D), Wqr, br)
        xt, xp = xt_new, xp_new

    pred = _pred_call(xt, xp, ls, ld)
    return pred[:E_LBL]


# speedup vs baseline: 5.6657x; 1.1142x over previous
import functools

import jax
import jax.numpy as jnp
from jax import lax
from jax.experimental import pallas as pl
from jax.experimental.pallas import tpu as pltpu
from jax.experimental.pallas import tpu_sc as plsc

N_TRACK = 50000
N_PLAYLIST = 10000
E = 600000
E_LBL = 100000
D = 128
DQ = 32

CHUNK = 1024
GPI = 128
EPAD = 622592
CHUNKS_B = 38
LPAD = 102400

QACC = 50048
PACC = 10240


def _zero_acc(zblk, rbuf, acc, s, rows_per_sub, blk):
    pltpu.sync_copy(zblk, rbuf.at[pl.ds(0, blk)])
    n_full = rows_per_sub // blk
    rem = rows_per_sub - n_full * blk
    base = s * rows_per_sub
    for j in range(n_full):
        pltpu.sync_copy(rbuf.at[pl.ds(0, blk)],
                        acc.at[pl.ds(base + j * blk, blk)])
    if rem:
        pltpu.sync_copy(rbuf.at[pl.ds(0, rem)],
                        acc.at[pl.ds(base + n_full * blk, rem)])


def _gs_chunk(table, acc, gbuf, dbuf, rbuf, sem_g, sem_s):
    prev = [None, None]
    for h in range(8):
        p = h % 2
        if prev[p] is not None:
            prev[p].wait()
        g = pltpu.async_copy(table.at[gbuf.at[h]],
                             rbuf.at[pl.ds(p * GPI, GPI)], sem_g)
        g.wait()
        prev[p] = pltpu.async_copy(rbuf.at[pl.ds(p * GPI, GPI)],
                                   acc.at[dbuf.at[h]], sem_s, add=True)
    for p in range(2):
        if prev[p] is not None:
            prev[p].wait()


def _seg_q_call(tableq, gidx2, didx2, zblkq, n_table, n_acc):
    mesh = plsc.VectorSubcoreMesh(core_axis_name="c", subcore_axis_name="s")

    st = (n_table // 16 + 7) // 8 * 8
    st_last = n_table - 15 * st
    rows_per_sub = n_acc // 16

    @functools.partial(
        pl.kernel, mesh=mesh,
        compiler_params=pltpu.CompilerParams(use_tc_tiling_on_sc=False),
        out_type=jax.ShapeDtypeStruct((4 * n_acc, DQ), jnp.float32),
        scratch_types=[
            pltpu.VMEM((8, GPI), jnp.int32),
            pltpu.VMEM((8, GPI), jnp.int32),
            pltpu.VMEM((256, DQ), jnp.float32),
            pltpu.VMEM_SHARED((n_table, DQ), jnp.float32),
            pltpu.VMEM_SHARED((n_acc, DQ), jnp.float32),
            pltpu.SemaphoreType.DMA,
            pltpu.SemaphoreType.DMA,
        ],
    )
    def k(table_h, gidx_h, didx_h, zblk_h, out_h,
          gbuf, dbuf, rbuf, table_s, acc, sem_g, sem_s):
        c = lax.axis_index("c")
        s = lax.axis_index("s")

        for b in range(2):
            q = 2 * c + b

            @pl.when(s < 15)
            def _stage():
                pltpu.sync_copy(
                    table_h.at[pl.ds(q * n_table + s * st, st)],
                    table_s.at[pl.ds(s * st, st)])
            @pl.when(s == 15)
            def _stage_tail():
                pltpu.sync_copy(
                    table_h.at[pl.ds(q * n_table + 15 * st, st_last)],
                    table_s.at[pl.ds(15 * st, st_last)])
            _zero_acc(zblk_h, rbuf, acc, s, rows_per_sub, 256)
            plsc.subcore_barrier()

            def body(i, carry):
                row0 = (s * CHUNKS_B + i) * 8
                pltpu.sync_copy(gidx_h.at[pl.ds(row0, 8)], gbuf)
                pltpu.sync_copy(didx_h.at[pl.ds(row0, 8)], dbuf)
                _gs_chunk(table_s, acc, gbuf, dbuf, rbuf, sem_g, sem_s)
                return carry

            lax.fori_loop(0, CHUNKS_B, body, 0)
            plsc.subcore_barrier()

            base = s * rows_per_sub
            n_full = rows_per_sub // 256
            rem = rows_per_sub - n_full * 256
            for j in range(n_full):
                pltpu.sync_copy(acc.at[pl.ds(base + j * 256, 256)], rbuf)
                pltpu.sync_copy(
                    rbuf, out_h.at[pl.ds(q * n_acc + base + j * 256, 256)])
            if rem:
                pltpu.sync_copy(acc.at[pl.ds(base + n_full * 256, rem)],
                                rbuf.at[pl.ds(0, rem)])
                pltpu.sync_copy(
                    rbuf.at[pl.ds(0, rem)],
                    out_h.at[pl.ds(q * n_acc + base + n_full * 256, rem)])
            plsc.subcore_barrier()

    return k(tableq, gidx2, didx2, zblkq)


def _pred_call(xt, xp, lsrc, ldst):
    mesh = plsc.VectorSubcoreMesh(core_axis_name="c", subcore_axis_name="s")

    @functools.partial(
        pl.kernel, mesh=mesh,
        compiler_params=pltpu.CompilerParams(needs_layout_passes=False),
        out_type=jax.ShapeDtypeStruct((LPAD,), jnp.float32),
        scratch_types=[
            pltpu.VMEM((GPI,), jnp.int32),
            pltpu.VMEM((GPI,), jnp.int32),
            pltpu.VMEM((GPI, D), jnp.float32),
            pltpu.VMEM((GPI, D), jnp.float32),
            pltpu.VMEM((GPI,), jnp.float32),
            pltpu.SemaphoreType.DMA,
        ],
    )
    def k(xt_h, xp_h, ls_h, ld_h, out_h, sbuf, dbuf, ra, rb, obuf, sem):
        c = lax.axis_index("c")
        s = lax.axis_index("s")
        wid = c * 16 + s

        def body(i, carry):
            row = wid * 25 + i
            pltpu.sync_copy(ls_h.at[pl.ds(row * GPI, GPI)], sbuf)
            pltpu.sync_copy(ld_h.at[pl.ds(row * GPI, GPI)], dbuf)
            ha = pltpu.make_async_copy(xt_h.at[sbuf], ra, sem)
            hb = pltpu.make_async_copy(xp_h.at[dbuf], rb, sem)
            ha.start(); hb.start()
            ha.wait(); hb.wait()

            lane = lax.iota(jnp.int32, 16)

            def dot_group(g, carry2):
                accvec = jnp.zeros((16,), jnp.float32)
                for rr in range(16):
                    r = g * 16 + rr
                    acc = ra[r, pl.ds(0, 16)] * rb[r, pl.ds(0, 16)]
                    for kk in range(1, 8):
                        acc = acc + (ra[r, pl.ds(kk * 16, 16)]
                                     * rb[r, pl.ds(kk * 16, 16)])
                    accvec = jnp.where(lane == rr, jnp.sum(acc), accvec)
                obuf[pl.ds(g * 16, 16)] = accvec
                return carry2

            lax.fori_loop(0, 8, dot_group, 0)
            pltpu.sync_copy(obuf, out_h.at[pl.ds(row * GPI, GPI)])
            return carry

        lax.fori_loop(0, 25, body, 0)

    return k(xt, xp, lsrc, ldst)


_RB = 2000


def _lin1(x, w, b):
    n = x.shape[0]

    def body(x_r, w_r, b_r, o_r, oq_r):
        mm = jax.lax.dot_general(
            x_r[...], w_r[...], (((1,), (0,)), ((), ())),
            preferred_element_type=jnp.float32) + b_r[...]
        o_r[...] = mm
        for q in range(4):
            oq_r[q] = mm[:, DQ * q:DQ * (q + 1)]

    return pl.pallas_call(
        body,
        grid=(n // _RB,),
        in_specs=[
            pl.BlockSpec((_RB, D), lambda i: (i, 0)),
            pl.BlockSpec((D, D), lambda i: (0, 0)),
            pl.BlockSpec((1, D), lambda i: (0, 0)),
        ],
        out_specs=[
            pl.BlockSpec((_RB, D), lambda i: (i, 0)),
            pl.BlockSpec((4, _RB, DQ), lambda i: (0, i, 0)),
        ],
        out_shape=[
            jax.ShapeDtypeStruct((n, D), jnp.float32),
            jax.ShapeDtypeStruct((4, n, DQ), jnp.float32),
        ],
    )(x, w, b.reshape(1, D))


def _comb2q(aggq, x, w4, w2, b):
    n = x.shape[0]

    def body(a_r, x_r, w4_r, w2_r, b_r, o_r, oq_r):
        mm = jax.lax.dot_general(
            x_r[...], w2_r[...], (((1,), (0,)), ((), ())),
            preferred_element_type=jnp.float32)
        for q in range(4):
            mm = mm + jax.lax.dot_general(
                a_r[q], w4_r[q], (((1,), (0,)), ((), ())),
                preferred_element_type=jnp.float32)
        mm = mm + b_r[...]
        o_r[...] = mm
        for q in range(4):
            oq_r[q] = mm[:, DQ * q:DQ * (q + 1)]

    return pl.pallas_call(
        body,
        grid=(n // _RB,),
        in_specs=[
            pl.BlockSpec((4, _RB, DQ), lambda i: (0, i, 0)),
            pl.BlockSpec((_RB, D), lambda i: (i, 0)),
            pl.BlockSpec((4, DQ, D), lambda i: (0, 0, 0)),
            pl.BlockSpec((D, D), lambda i: (0, 0)),
            pl.BlockSpec((1, D), lambda i: (0, 0)),
        ],
        out_specs=[
            pl.BlockSpec((_RB, D), lambda i: (i, 0)),
            pl.BlockSpec((4, _RB, DQ), lambda i: (0, i, 0)),
        ],
        out_shape=[
            jax.ShapeDtypeStruct((n, D), jnp.float32),
            jax.ShapeDtypeStruct((4, n, DQ), jnp.float32),
        ],
    )(aggq, x, w4, w2, b.reshape(1, D))


def kernel(x_track, x_playlist, edge_src, edge_dst, label_src, label_dst,
           W_lt, b_lt, W_lp, b_lp,
           Wrel_c0, Wroot_c0, b_c0, Wrel_r0, Wroot_r0, b_r0,
           Wrel_c1, Wroot_c1, b_c1, Wrel_r1, Wroot_r1, b_r1):
    ep = EPAD - E
    g_p = jnp.pad(edge_src, (0, ep)).reshape(-1, GPI)
    s_p = jnp.pad(edge_dst, (0, ep),
                  constant_values=N_PLAYLIST).reshape(-1, GPI)
    g_t = jnp.pad(edge_dst, (0, ep)).reshape(-1, GPI)
    s_t = jnp.pad(edge_src, (0, ep), constant_values=N_TRACK).reshape(-1, GPI)
    lp = LPAD - E_LBL
    ls = jnp.pad(label_src, (0, lp))
    ld = jnp.pad(label_dst, (0, lp))
    zblkq = jnp.zeros((256, DQ), jnp.float32)

    xt, xtq = _lin1(x_track, W_lt, b_lt)
    xp, xpq = _lin1(x_playlist, W_lp, b_lp)

    for (Wrc, Wqc, bc, Wrr, Wqr, br) in (
            (Wrel_c0, Wroot_c0, b_c0, Wrel_r0, Wroot_r0, b_r0),
            (Wrel_c1, Wroot_c1, b_c1, Wrel_r1, Wroot_r1, b_r1)):
        agg_pq = _seg_q_call(xtq.reshape(4 * N_TRACK, DQ),
                             g_p, s_p, zblkq, N_TRACK, PACC)
        agg_tq = _seg_q_call(xpq.reshape(4 * N_PLAYLIST, DQ),
                             g_t, s_t, zblkq, N_PLAYLIST, QACC)
        xp, xpq = _comb2q(agg_pq.reshape(4, PACC, DQ), xp,
                          Wrc.reshape(4, DQ, D), Wqc, bc)
        xt, xtq = _comb2q(agg_tq.reshape(4, QACC, DQ), xt,
                          Wrr.reshape(4, DQ, D), Wqr, br)

    pred = _pred_call(xt, xp, ls, ld)
    return pred[:E_LBL]


# speedup vs baseline: 5.7783x; 1.0199x over previous
import functools

import jax
import jax.numpy as jnp
from jax import lax
from jax.experimental import pallas as pl
from jax.experimental.pallas import tpu as pltpu
from jax.experimental.pallas import tpu_sc as plsc

N_TRACK = 50000
N_PLAYLIST = 10000
E = 600000
E_LBL = 100000
D = 128
DQ = 32

CHUNK = 1024
GPI = 128
EPAD = 622592
CHUNKS_B = 38
LPAD = 102400

QACC = 50048
PACC = 10240


def _zero_acc(zblk, rbuf, acc, s, rows_per_sub, blk):
    pltpu.sync_copy(zblk, rbuf.at[pl.ds(0, blk)])
    n_full = rows_per_sub // blk
    rem = rows_per_sub - n_full * blk
    base = s * rows_per_sub
    for j in range(n_full):
        pltpu.sync_copy(rbuf.at[pl.ds(0, blk)],
                        acc.at[pl.ds(base + j * blk, blk)])
    if rem:
        pltpu.sync_copy(rbuf.at[pl.ds(0, rem)],
                        acc.at[pl.ds(base + n_full * blk, rem)])


def _gs_chunk(table, acc, ibuf, rbuf, sem_g, sem_s):
    sprev = [None, None]
    gh = [None, None]
    gh[0] = pltpu.async_copy(table.at[ibuf.at[0]],
                             rbuf.at[pl.ds(0, GPI)], sem_g)
    for h in range(8):
        p = h % 2
        o = 1 - p
        if h + 1 < 8:
            if sprev[o] is not None:
                sprev[o].wait()
            gh[o] = pltpu.async_copy(table.at[ibuf.at[h + 1]],
                                     rbuf.at[pl.ds(o * GPI, GPI)], sem_g)
        gh[p].wait()
        sprev[p] = pltpu.async_copy(rbuf.at[pl.ds(p * GPI, GPI)],
                                    acc.at[ibuf.at[8 + h]], sem_s, add=True)
    for p in range(2):
        if sprev[p] is not None:
            sprev[p].wait()


def _seg_q_call(tableq, idx2, zblkq, n_table, n_acc):
    mesh = plsc.VectorSubcoreMesh(core_axis_name="c", subcore_axis_name="s")

    st = (n_table // 16 + 7) // 8 * 8
    st_last = n_table - 15 * st
    rows_per_sub = n_acc // 16

    @functools.partial(
        pl.kernel, mesh=mesh,
        compiler_params=pltpu.CompilerParams(use_tc_tiling_on_sc=False),
        out_type=jax.ShapeDtypeStruct((4 * n_acc, DQ), jnp.float32),
        scratch_types=[
            pltpu.VMEM((16, GPI), jnp.int32),
            pltpu.VMEM((256, DQ), jnp.float32),
            pltpu.VMEM_SHARED((n_table, DQ), jnp.float32),
            pltpu.VMEM_SHARED((n_acc, DQ), jnp.float32),
            pltpu.SemaphoreType.DMA,
            pltpu.SemaphoreType.DMA,
        ],
    )
    def k(table_h, idx_h, zblk_h, out_h,
          ibuf, rbuf, table_s, acc, sem_g, sem_s):
        c = lax.axis_index("c")
        s = lax.axis_index("s")

        for b in range(2):
            q = 2 * c + b

            @pl.when(s < 15)
            def _stage():
                pltpu.sync_copy(
                    table_h.at[pl.ds(q * n_table + s * st, st)],
                    table_s.at[pl.ds(s * st, st)])
            @pl.when(s == 15)
            def _stage_tail():
                pltpu.sync_copy(
                    table_h.at[pl.ds(q * n_table + 15 * st, st_last)],
                    table_s.at[pl.ds(15 * st, st_last)])
            _zero_acc(zblk_h, rbuf, acc, s, rows_per_sub, 256)
            plsc.subcore_barrier()

            def body(i, carry):
                row0 = (s * CHUNKS_B + i) * 16
                pltpu.sync_copy(idx_h.at[pl.ds(row0, 16)], ibuf)
                _gs_chunk(table_s, acc, ibuf, rbuf, sem_g, sem_s)
                return carry

            lax.fori_loop(0, CHUNKS_B, body, 0)
            plsc.subcore_barrier()

            base = s * rows_per_sub
            n_full = rows_per_sub // 256
            rem = rows_per_sub - n_full * 256
            for j in range(n_full):
                pltpu.sync_copy(acc.at[pl.ds(base + j * 256, 256)], rbuf)
                pltpu.sync_copy(
                    rbuf, out_h.at[pl.ds(q * n_acc + base + j * 256, 256)])
            if rem:
                pltpu.sync_copy(acc.at[pl.ds(base + n_full * 256, rem)],
                                rbuf.at[pl.ds(0, rem)])
                pltpu.sync_copy(
                    rbuf.at[pl.ds(0, rem)],
                    out_h.at[pl.ds(q * n_acc + base + n_full * 256, rem)])
            plsc.subcore_barrier()

    return k(tableq, idx2, zblkq)


def _pred_call(xt, xp, lsrc, ldst):
    mesh = plsc.VectorSubcoreMesh(core_axis_name="c", subcore_axis_name="s")

    @functools.partial(
        pl.kernel, mesh=mesh,
        compiler_params=pltpu.CompilerParams(needs_layout_passes=False),
        out_type=jax.ShapeDtypeStruct((LPAD,), jnp.float32),
        scratch_types=[
            pltpu.VMEM((GPI,), jnp.int32),
            pltpu.VMEM((GPI,), jnp.int32),
            pltpu.VMEM((GPI, D), jnp.float32),
            pltpu.VMEM((GPI, D), jnp.float32),
            pltpu.VMEM((GPI,), jnp.float32),
            pltpu.SemaphoreType.DMA,
        ],
    )
    def k(xt_h, xp_h, ls_h, ld_h, out_h, sbuf, dbuf, ra, rb, obuf, sem):
        c = lax.axis_index("c")
        s = lax.axis_index("s")
        wid = c * 16 + s

        def body(i, carry):
            row = wid * 25 + i
            pltpu.sync_copy(ls_h.at[pl.ds(row * GPI, GPI)], sbuf)
            pltpu.sync_copy(ld_h.at[pl.ds(row * GPI, GPI)], dbuf)
            ha = pltpu.make_async_copy(xt_h.at[sbuf], ra, sem)
            hb = pltpu.make_async_copy(xp_h.at[dbuf], rb, sem)
            ha.start(); hb.start()
            ha.wait(); hb.wait()

            lane = lax.iota(jnp.int32, 16)

            def dot_group(g, carry2):
                accvec = jnp.zeros((16,), jnp.float32)
                for rr in range(16):
                    r = g * 16 + rr
                    acc = ra[r, pl.ds(0, 16)] * rb[r, pl.ds(0, 16)]
                    for kk in range(1, 8):
                        acc = acc + (ra[r, pl.ds(kk * 16, 16)]
                                     * rb[r, pl.ds(kk * 16, 16)])
                    accvec = jnp.where(lane == rr, jnp.sum(acc), accvec)
                obuf[pl.ds(g * 16, 16)] = accvec
                return carry2

            lax.fori_loop(0, 8, dot_group, 0)
            pltpu.sync_copy(obuf, out_h.at[pl.ds(row * GPI, GPI)])
            return carry

        lax.fori_loop(0, 25, body, 0)

    return k(xt, xp, lsrc, ldst)


_RB = 2000


def _lin1(x, w, b):
    n = x.shape[0]

    def body(x_r, w_r, b_r, o_r, oq_r):
        mm = jax.lax.dot_general(
            x_r[...], w_r[...], (((1,), (0,)), ((), ())),
            preferred_element_type=jnp.float32) + b_r[...]
        o_r[...] = mm
        for q in range(4):
            oq_r[q] = mm[:, DQ * q:DQ * (q + 1)]

    return pl.pallas_call(
        body,
        grid=(n // _RB,),
        in_specs=[
            pl.BlockSpec((_RB, D), lambda i: (i, 0)),
            pl.BlockSpec((D, D), lambda i: (0, 0)),
            pl.BlockSpec((1, D), lambda i: (0, 0)),
        ],
        out_specs=[
            pl.BlockSpec((_RB, D), lambda i: (i, 0)),
            pl.BlockSpec((4, _RB, DQ), lambda i: (0, i, 0)),
        ],
        out_shape=[
            jax.ShapeDtypeStruct((n, D), jnp.float32),
            jax.ShapeDtypeStruct((4, n, DQ), jnp.float32),
        ],
    )(x, w, b.reshape(1, D))


def _comb2q(aggq, x, w4, w2, b):
    n = x.shape[0]

    def body(a_r, x_r, w4_r, w2_r, b_r, o_r, oq_r):
        mm = jax.lax.dot_general(
            x_r[...], w2_r[...], (((1,), (0,)), ((), ())),
            preferred_element_type=jnp.float32)
        for q in range(4):
            mm = mm + jax.lax.dot_general(
                a_r[q], w4_r[q], (((1,), (0,)), ((), ())),
                preferred_element_type=jnp.float32)
        mm = mm + b_r[...]
        o_r[...] = mm
        for q in range(4):
            oq_r[q] = mm[:, DQ * q:DQ * (q + 1)]

    return pl.pallas_call(
        body,
        grid=(n // _RB,),
        in_specs=[
            pl.BlockSpec((4, _RB, DQ), lambda i: (0, i, 0)),
            pl.BlockSpec((_RB, D), lambda i: (i, 0)),
            pl.BlockSpec((4, DQ, D), lambda i: (0, 0, 0)),
            pl.BlockSpec((D, D), lambda i: (0, 0)),
            pl.BlockSpec((1, D), lambda i: (0, 0)),
        ],
        out_specs=[
            pl.BlockSpec((_RB, D), lambda i: (i, 0)),
            pl.BlockSpec((4, _RB, DQ), lambda i: (0, i, 0)),
        ],
        out_shape=[
            jax.ShapeDtypeStruct((n, D), jnp.float32),
            jax.ShapeDtypeStruct((4, n, DQ), jnp.float32),
        ],
    )(aggq, x, w4, w2, b.reshape(1, D))


def kernel(x_track, x_playlist, edge_src, edge_dst, label_src, label_dst,
           W_lt, b_lt, W_lp, b_lp,
           Wrel_c0, Wroot_c0, b_c0, Wrel_r0, Wroot_r0, b_r0,
           Wrel_c1, Wroot_c1, b_c1, Wrel_r1, Wroot_r1, b_r1):
    ep = EPAD - E

    def interleave(g, d):
        return jnp.concatenate(
            [g.reshape(-1, 8, GPI), d.reshape(-1, 8, GPI)],
            axis=1).reshape(-1, GPI)

    i_p = interleave(jnp.pad(edge_src, (0, ep)),
                     jnp.pad(edge_dst, (0, ep),
                             constant_values=N_PLAYLIST))
    i_t = interleave(jnp.pad(edge_dst, (0, ep)),
                     jnp.pad(edge_src, (0, ep), constant_values=N_TRACK))
    lp = LPAD - E_LBL
    ls = jnp.pad(label_src, (0, lp))
    ld = jnp.pad(label_dst, (0, lp))
    zblkq = jnp.zeros((256, DQ), jnp.float32)

    xt, xtq = _lin1(x_track, W_lt, b_lt)
    xp, xpq = _lin1(x_playlist, W_lp, b_lp)

    for (Wrc, Wqc, bc, Wrr, Wqr, br) in (
            (Wrel_c0, Wroot_c0, b_c0, Wrel_r0, Wroot_r0, b_r0),
            (Wrel_c1, Wroot_c1, b_c1, Wrel_r1, Wroot_r1, b_r1)):
        agg_pq = _seg_q_call(xtq.reshape(4 * N_TRACK, DQ),
                             i_p, zblkq, N_TRACK, PACC)
        agg_tq = _seg_q_call(xpq.reshape(4 * N_PLAYLIST, DQ),
                             i_t, zblkq, N_PLAYLIST, QACC)
        xp, xpq = _comb2q(agg_pq.reshape(4, PACC, DQ), xp,
                          Wrc.reshape(4, DQ, D), Wqc, bc)
        xt, xtq = _comb2q(agg_tq.reshape(4, QACC, DQ), xt,
                          Wrr.reshape(4, DQ, D), Wqr, br)

    pred = _pred_call(xt, xp, ls, ld)
    return pred[:E_LBL]


# speedup vs baseline: 6.1428x; 1.0631x over previous
import functools

import jax
import jax.numpy as jnp
from jax import lax
from jax.experimental import pallas as pl
from jax.experimental.pallas import tpu as pltpu
from jax.experimental.pallas import tpu_sc as plsc

N_TRACK = 50000
N_PLAYLIST = 10000
E = 600000
E_LBL = 100000
D = 128
DQ = 32

CHUNK = 1024
GPI = 128
EPAD = 622592
CHUNKS_B = 38
LPAD = 102400

QACC = 50048
PACC = 10240


def _zero_acc(zblk, rbuf, acc, s, rows_per_sub, blk):
    pltpu.sync_copy(zblk, rbuf.at[pl.ds(0, blk)])
    n_full = rows_per_sub // blk
    rem = rows_per_sub - n_full * blk
    base = s * rows_per_sub
    for j in range(n_full):
        pltpu.sync_copy(rbuf.at[pl.ds(0, blk)],
                        acc.at[pl.ds(base + j * blk, blk)])
    if rem:
        pltpu.sync_copy(rbuf.at[pl.ds(0, rem)],
                        acc.at[pl.ds(base + n_full * blk, rem)])


def _gs_chunk(table, acc, ibuf, rbuf, sem_g, sem_s):
    sprev = [None, None]
    gh = [None, None]
    gh[0] = pltpu.async_copy(table.at[ibuf.at[0]],
                             rbuf.at[pl.ds(0, GPI)], sem_g)
    for h in range(8):
        p = h % 2
        o = 1 - p
        if h + 1 < 8:
            if sprev[o] is not None:
                sprev[o].wait()
            gh[o] = pltpu.async_copy(table.at[ibuf.at[h + 1]],
                                     rbuf.at[pl.ds(o * GPI, GPI)], sem_g)
        gh[p].wait()
        sprev[p] = pltpu.async_copy(rbuf.at[pl.ds(p * GPI, GPI)],
                                    acc.at[ibuf.at[8 + h]], sem_s, add=True)
    for p in range(2):
        if sprev[p] is not None:
            sprev[p].wait()


def _seg_q_call(tableq, idx2, zblkq, n_table, n_acc):
    mesh = plsc.VectorSubcoreMesh(core_axis_name="c", subcore_axis_name="s")

    st = (n_table // 16 + 7) // 8 * 8
    st_last = n_table - 15 * st
    rows_per_sub = n_acc // 16

    @functools.partial(
        pl.kernel, mesh=mesh,
        compiler_params=pltpu.CompilerParams(use_tc_tiling_on_sc=False),
        out_type=jax.ShapeDtypeStruct((4 * n_acc, DQ), jnp.float32),
        scratch_types=[
            pltpu.VMEM((16, GPI), jnp.int32),
            pltpu.VMEM((256, DQ), jnp.float32),
            pltpu.VMEM_SHARED((n_table, DQ), jnp.float32),
            pltpu.VMEM_SHARED((n_acc, DQ), jnp.float32),
            pltpu.SemaphoreType.DMA,
            pltpu.SemaphoreType.DMA,
        ],
    )
    def k(table_h, idx_h, zblk_h, out_h,
          ibuf, rbuf, table_s, acc, sem_g, sem_s):
        c = lax.axis_index("c")
        s = lax.axis_index("s")

        for b in range(2):
            q = 2 * c + b

            @pl.when(s < 15)
            def _stage():
                pltpu.sync_copy(
                    table_h.at[pl.ds(q * n_table + s * st, st)],
                    table_s.at[pl.ds(s * st, st)])
            @pl.when(s == 15)
            def _stage_tail():
                pltpu.sync_copy(
                    table_h.at[pl.ds(q * n_table + 15 * st, st_last)],
                    table_s.at[pl.ds(15 * st, st_last)])
            _zero_acc(zblk_h, rbuf, acc, s, rows_per_sub, 256)
            plsc.subcore_barrier()

            def body(i, carry):
                row0 = (s * CHUNKS_B + i) * 16
                pltpu.sync_copy(idx_h.at[pl.ds(row0, 16)], ibuf)
                _gs_chunk(table_s, acc, ibuf, rbuf, sem_g, sem_s)
                return carry

            lax.fori_loop(0, CHUNKS_B, body, 0)
            plsc.subcore_barrier()

            base = s * rows_per_sub
            n_full = rows_per_sub // 256
            rem = rows_per_sub - n_full * 256
            for j in range(n_full):
                pltpu.sync_copy(acc.at[pl.ds(base + j * 256, 256)], rbuf)
                pltpu.sync_copy(
                    rbuf, out_h.at[pl.ds(q * n_acc + base + j * 256, 256)])
            if rem:
                pltpu.sync_copy(acc.at[pl.ds(base + n_full * 256, rem)],
                                rbuf.at[pl.ds(0, rem)])
                pltpu.sync_copy(
                    rbuf.at[pl.ds(0, rem)],
                    out_h.at[pl.ds(q * n_acc + base + n_full * 256, rem)])
            plsc.subcore_barrier()

    return k(tableq, idx2, zblkq)


def _pred_call(xt, xp, lsd):
    mesh = plsc.VectorSubcoreMesh(core_axis_name="c", subcore_axis_name="s")

    @functools.partial(
        pl.kernel, mesh=mesh,
        compiler_params=pltpu.CompilerParams(needs_layout_passes=False),
        out_type=jax.ShapeDtypeStruct((LPAD,), jnp.float32),
        scratch_types=[
            pltpu.VMEM((2, 2, GPI), jnp.int32),
            pltpu.VMEM((2, GPI, D), jnp.float32),
            pltpu.VMEM((64, D), jnp.float32),
            pltpu.VMEM((GPI,), jnp.float32),
            pltpu.VMEM_SHARED((N_PLAYLIST, D), jnp.float32),
            pltpu.SemaphoreType.DMA,
            pltpu.SemaphoreType.DMA,
        ],
    )
    def k(xt_h, xp_h, lsd_h, out_h, ibuf, ra, rb, obuf, xps, sem_a, sem_b):
        c = lax.axis_index("c")
        s = lax.axis_index("s")
        wid = c * 16 + s

        @pl.when(s < 15)
        def _stage():
            pltpu.sync_copy(xp_h.at[pl.ds(s * 640, 640)],
                            xps.at[pl.ds(s * 640, 640)])
        @pl.when(s == 15)
        def _stage_tail():
            pltpu.sync_copy(xp_h.at[pl.ds(9600, 400)],
                            xps.at[pl.ds(9600, 400)])
        plsc.subcore_barrier()

        lane = lax.iota(jnp.int32, 16)

        def compute(p, row):
            for hh in range(2):
                pltpu.async_copy(xps.at[ibuf.at[p, 1, pl.ds(hh * 64, 64)]],
                                 rb, sem_b).wait()

                def dot_group(g, carry2, hh=hh):
                    def dot_row(rr, accvec):
                        r = g * 16 + rr
                        acc = (ra[p, hh * 64 + r, pl.ds(0, 16)]
                               * rb[r, pl.ds(0, 16)])
                        for kk in range(1, 8):
                            acc = acc + (ra[p, hh * 64 + r, pl.ds(kk * 16, 16)]
                                         * rb[r, pl.ds(kk * 16, 16)])
                        return jnp.where(lane == rr, jnp.sum(acc), accvec)

                    accvec = lax.fori_loop(0, 16, dot_row,
                                           jnp.zeros((16,), jnp.float32))
                    obuf[pl.ds(hh * 64 + g * 16, 16)] = accvec
                    return carry2

                lax.fori_loop(0, 4, dot_group, 0)
            pltpu.sync_copy(obuf, out_h.at[pl.ds(row * GPI, GPI)])

        def load_idx(p, row):
            pltpu.sync_copy(lsd_h.at[pl.ds(2 * row, 2)], ibuf.at[p])

        def start_gather(p):
            return pltpu.async_copy(xt_h.at[ibuf.at[p, 0]], ra.at[p], sem_a)

        base = wid * 25
        load_idx(0, base)
        g0 = start_gather(0)
        gh = [g0, None]

        def pair(k2, carry):
            i0 = base + 2 * k2
            for p in range(2):
                i = i0 + p
                gh[p].wait()
                nxt = jnp.minimum(i + 1, base + 24)
                load_idx(1 - p, nxt)
                gh[1 - p] = start_gather(1 - p)
                compute(p, i)
            return carry

        lax.fori_loop(0, 12, pair, 0)
        gh[0].wait()
        compute(0, base + 24)

    return k(xt, xp, lsd)


_RB = 2000


def _lin1(x, w, b):
    n = x.shape[0]

    def body(x_r, w_r, b_r, o_r, oq_r):
        mm = jax.lax.dot_general(
            x_r[...], w_r[...], (((1,), (0,)), ((), ())),
            preferred_element_type=jnp.float32) + b_r[...]
        o_r[...] = mm
        for q in range(4):
            oq_r[q] = mm[:, DQ * q:DQ * (q + 1)]

    return pl.pallas_call(
        body,
        grid=(n // _RB,),
        in_specs=[
            pl.BlockSpec((_RB, D), lambda i: (i, 0)),
            pl.BlockSpec((D, D), lambda i: (0, 0)),
            pl.BlockSpec((1, D), lambda i: (0, 0)),
        ],
        out_specs=[
            pl.BlockSpec((_RB, D), lambda i: (i, 0)),
            pl.BlockSpec((4, _RB, DQ), lambda i: (0, i, 0)),
        ],
        out_shape=[
            jax.ShapeDtypeStruct((n, D), jnp.float32),
            jax.ShapeDtypeStruct((4, n, DQ), jnp.float32),
        ],
    )(x, w, b.reshape(1, D))


def _comb2q(aggq, x, w4, w2, b):
    n = x.shape[0]

    def body(a_r, x_r, w4_r, w2_r, b_r, o_r, oq_r):
        mm = jax.lax.dot_general(
            x_r[...], w2_r[...], (((1,), (0,)), ((), ())),
            preferred_element_type=jnp.float32)
        for q in range(4):
            mm = mm + jax.lax.dot_general(
                a_r[q], w4_r[q], (((1,), (0,)), ((), ())),
                preferred_element_type=jnp.float32)
        mm = mm + b_r[...]
        o_r[...] = mm
        for q in range(4):
            oq_r[q] = mm[:, DQ * q:DQ * (q + 1)]

    return pl.pallas_call(
        body,
        grid=(n // _RB,),
        in_specs=[
            pl.BlockSpec((4, _RB, DQ), lambda i: (0, i, 0)),
            pl.BlockSpec((_RB, D), lambda i: (i, 0)),
            pl.BlockSpec((4, DQ, D), lambda i: (0, 0, 0)),
            pl.BlockSpec((D, D), lambda i: (0, 0)),
            pl.BlockSpec((1, D), lambda i: (0, 0)),
        ],
        out_specs=[
            pl.BlockSpec((_RB, D), lambda i: (i, 0)),
            pl.BlockSpec((4, _RB, DQ), lambda i: (0, i, 0)),
        ],
        out_shape=[
            jax.ShapeDtypeStruct((n, D), jnp.float32),
            jax.ShapeDtypeStruct((4, n, DQ), jnp.float32),
        ],
    )(aggq, x, w4, w2, b.reshape(1, D))


def kernel(x_track, x_playlist, edge_src, edge_dst, label_src, label_dst,
           W_lt, b_lt, W_lp, b_lp,
           Wrel_c0, Wroot_c0, b_c0, Wrel_r0, Wroot_r0, b_r0,
           Wrel_c1, Wroot_c1, b_c1, Wrel_r1, Wroot_r1, b_r1):
    ep = EPAD - E

    def interleave(g, d):
        return jnp.concatenate(
            [g.reshape(-1, 8, GPI), d.reshape(-1, 8, GPI)],
            axis=1).reshape(-1, GPI)

    i_p = interleave(jnp.pad(edge_src, (0, ep)),
                     jnp.pad(edge_dst, (0, ep),
                             constant_values=N_PLAYLIST))
    i_t = interleave(jnp.pad(edge_dst, (0, ep)),
                     jnp.pad(edge_src, (0, ep), constant_values=N_TRACK))
    lp = LPAD - E_LBL
    lsd = jnp.concatenate(
        [jnp.pad(label_src, (0, lp)).reshape(-1, 1, GPI),
         jnp.pad(label_dst, (0, lp)).reshape(-1, 1, GPI)],
        axis=1).reshape(-1, GPI)
    zblkq = jnp.zeros((256, DQ), jnp.float32)

    xt, xtq = _lin1(x_track, W_lt, b_lt)
    xp, xpq = _lin1(x_playlist, W_lp, b_lp)

    for (Wrc, Wqc, bc, Wrr, Wqr, br) in (
            (Wrel_c0, Wroot_c0, b_c0, Wrel_r0, Wroot_r0, b_r0),
            (Wrel_c1, Wroot_c1, b_c1, Wrel_r1, Wroot_r1, b_r1)):
        agg_pq = _seg_q_call(xtq.reshape(4 * N_TRACK, DQ),
                             i_p, zblkq, N_TRACK, PACC)
        agg_tq = _seg_q_call(xpq.reshape(4 * N_PLAYLIST, DQ),
                             i_t, zblkq, N_PLAYLIST, QACC)
        xp, xpq = _comb2q(agg_pq.reshape(4, PACC, DQ), xp,
                          Wrc.reshape(4, DQ, D), Wqc, bc)
        xt, xtq = _comb2q(agg_tq.reshape(4, QACC, DQ), xt,
                          Wrr.reshape(4, DQ, D), Wqr, br)

    pred = _pred_call(xt, xp, lsd)
    return pred[:E_LBL]
